# baseline (speedup 1.0000x reference)
import jax
import jax.numpy as jnp
from jax.experimental import pallas as pl
from jax.experimental.pallas import tpu as pltpu

_ROWS_PER_BLOCK = 24


def _copy_body(x1_ref, x2_ref, o1_ref, o2_ref):
    o1_ref[...] = x1_ref[...]
    o2_ref[...] = x2_ref[...]


def kernel(x1, x2):
    N, C, H, W = x1.shape
    rows = N * C
    a = x1.reshape(rows, H, W)
    b = x2.reshape(rows, H, W)
    grid = (rows // _ROWS_PER_BLOCK,)
    spec = pl.BlockSpec((_ROWS_PER_BLOCK, H, W), lambda i: (i, 0, 0))
    out1, out2 = pl.pallas_call(
        _copy_body,
        grid=grid,
        out_shape=(
            jax.ShapeDtypeStruct((rows, H, W), x1.dtype),
            jax.ShapeDtypeStruct((rows, H, W), x2.dtype),
        ),
        in_specs=[spec, spec],
        out_specs=(spec, spec),
        compiler_params=pltpu.CompilerParams(
            dimension_semantics=("parallel",),
            vmem_limit_bytes=128 * 1024 * 1024,
        ),
    )(a, b)
    return (out1.reshape(N, C, H, W), out2.reshape(N, C, H, W))


# speedup vs baseline: 1.0037x; 1.0037x over previous
import jax
import jax.numpy as jnp
from jax.experimental import pallas as pl
from jax.experimental.pallas import tpu as pltpu

_ROWS_PER_BLOCK = 64


def _copy_body(x_ref, o_ref):
    o_ref[...] = x_ref[...]


def _copy_one(a):
    rows, H, W = a.shape
    spec = pl.BlockSpec((_ROWS_PER_BLOCK, H, W), lambda i: (i, 0, 0))
    return pl.pallas_call(
        _copy_body,
        grid=(rows // _ROWS_PER_BLOCK,),
        out_shape=jax.ShapeDtypeStruct((rows, H, W), a.dtype),
        in_specs=[spec],
        out_specs=spec,
        compiler_params=pltpu.CompilerParams(
            dimension_semantics=("parallel",),
        ),
    )(a)


def kernel(x1, x2):
    N, C, H, W = x1.shape
    rows = N * C
    out1 = _copy_one(x1.reshape(rows, H, W))
    out2 = _copy_one(x2.reshape(rows, H, W))
    return (out1.reshape(N, C, H, W), out2.reshape(N, C, H, W))
